# baseline (speedup 1.0000x reference)
import functools

import jax
import jax.numpy as jnp
from jax import lax
from jax.experimental import pallas as pl
from jax.experimental.pallas import tpu as pltpu
from jax.experimental.pallas import tpu_sc as plsc

N_NODES = 10000
DIM = 128
N_PAD = 10240
K_EDGE = 64
C_CHUNKS = 160
NC, NS = 2, 16
E_PAD = NC * NS * C_CHUNKS * K_EDGE
ROWS_PER_TILE = N_PAD // NS
GSZ = 16
C_GROUPS = C_CHUNKS // GSZ
NGRP_PAD = C_GROUPS + 1
NBUF = 4


def _sc_aggregate_body(h_hbm, srcs_hbm, dsts_hbm, out_hbm,
                       sidx, didx, rows, acc, sem_g, sem_s, sem_i, sem_h):
    c = lax.axis_index("c")
    s = lax.axis_index("s")
    row0 = pl.multiple_of(s * ROWS_PER_TILE, ROWS_PER_TILE)
    wid = c * NS + s
    gbase = wid * NGRP_PAD

    def wait_g():
        pltpu.make_async_copy(h_hbm.at[sidx.at[0, 0]], rows.at[0], sem_g).wait()

    def wait_s():
        pltpu.make_async_copy(rows.at[0], acc.at[pl.ds(0, K_EDGE)],
                              sem_s).wait()

    def load_idx(g, b):
        pltpu.async_copy(srcs_hbm.at[g], sidx.at[b], sem_i)
        pltpu.async_copy(dsts_hbm.at[g], didx.at[b], sem_i)

    def wait_i():
        pltpu.make_async_copy(srcs_hbm.at[0], sidx.at[0], sem_i).wait()
        pltpu.make_async_copy(dsts_hbm.at[0], didx.at[0], sem_i).wait()

    def gather(b, j, rb):
        pltpu.async_copy(h_hbm.at[sidx.at[b, j]], rows.at[rb], sem_g)

    def scatter(b, j, rb):
        pltpu.async_copy(rows.at[rb], acc.at[didx.at[b, j]], sem_s, add=True)

    pltpu.async_copy(h_hbm.at[pl.ds(row0, ROWS_PER_TILE)],
                     acc.at[pl.ds(row0, ROWS_PER_TILE)], sem_h)
    pltpu.sync_copy(srcs_hbm.at[gbase], sidx.at[0])
    pltpu.sync_copy(dsts_hbm.at[gbase], didx.at[0])
    load_idx(gbase + 1, 1)
    gather(0, 0, 0)
    gather(0, 1, 1)
    pltpu.make_async_copy(h_hbm.at[pl.ds(row0, ROWS_PER_TILE)],
                          acc.at[pl.ds(row0, ROWS_PER_TILE)], sem_h).wait()
    plsc.subcore_barrier()

    def run_group(g_load, b, n_is_small):
        for j in range(GSZ):
            wait_g()
            scatter(b, j, j % NBUF)
            if not (n_is_small and j < 2):
                wait_s()
            if j < GSZ - 2:
                gather(b, j + 2, (j + 2) % NBUF)
            else:
                if j == GSZ - 2:
                    wait_i()
                gather(1 - b, j - (GSZ - 2), (j + 2) % NBUF)
        load_idx(g_load, b)

    run_group(gbase + 2, 0, True)

    def pair_body(k, _):
        g = 2 * k + 1
        run_group(gbase + g + 2, 1, False)
        run_group(gbase + g + 3, 0, False)
        return 0

    lax.fori_loop(0, (C_GROUPS - 2) // 2, pair_body, 0)

    wait_i()
    for j in range(GSZ):
        wait_g()
        scatter(1, j, j % NBUF)
        wait_s()
        if j < GSZ - 2:
            gather(1, j + 2, (j + 2) % NBUF)
    wait_s()
    wait_s()
    plsc.subcore_barrier()

    pltpu.sync_copy(acc.at[pl.ds(row0, ROWS_PER_TILE)],
                    out_hbm.at[c, pl.ds(row0, ROWS_PER_TILE)])


_sc_aggregate = functools.partial(
    pl.kernel,
    out_type=jax.ShapeDtypeStruct((NC, N_PAD, DIM), jnp.float32),
    mesh=plsc.VectorSubcoreMesh(core_axis_name="c", subcore_axis_name="s"),
    scratch_types=[
        pltpu.VMEM((2, GSZ, K_EDGE), jnp.int32),
        pltpu.VMEM((2, GSZ, K_EDGE), jnp.int32),
        pltpu.VMEM((NBUF, K_EDGE, DIM), jnp.float32),
        pltpu.VMEM_SHARED((N_PAD, DIM), jnp.float32),
        pltpu.SemaphoreType.DMA,
        pltpu.SemaphoreType.DMA,
        pltpu.SemaphoreType.DMA,
        pltpu.SemaphoreType.DMA,
    ],
)(_sc_aggregate_body)


def _mlp_body(final_relu, p_ref, h_ref, wa_ref, ba_ref, wb_ref, bb_ref, o_ref):
    z = p_ref[0] + p_ref[1] - h_ref[...]
    y = jnp.maximum(
        jnp.dot(z, wa_ref[...], preferred_element_type=jnp.float32)
        + ba_ref[...], 0.0)
    y = jnp.dot(y, wb_ref[...], preferred_element_type=jnp.float32) + bb_ref[...]
    if final_relu:
        y = jnp.maximum(y, 0.0)
    o_ref[...] = y


def _tc_mlp(p, h, wa, ba, wb, bb, final_relu):
    rb = 1280
    grid = N_PAD // rb
    return pl.pallas_call(
        functools.partial(_mlp_body, final_relu),
        grid=(grid,),
        in_specs=[
            pl.BlockSpec((NC, rb, DIM), lambda i: (0, i, 0)),
            pl.BlockSpec((rb, DIM), lambda i: (i, 0)),
            pl.BlockSpec((DIM, DIM), lambda i: (0, 0)),
            pl.BlockSpec((1, DIM), lambda i: (0, 0)),
            pl.BlockSpec((DIM, DIM), lambda i: (0, 0)),
            pl.BlockSpec((1, DIM), lambda i: (0, 0)),
        ],
        out_specs=pl.BlockSpec((rb, DIM), lambda i: (i, 0)),
        out_shape=jax.ShapeDtypeStruct((N_PAD, DIM), jnp.float32),
    )(p, h, wa, ba, wb, bb)


def kernel(x, edge_index, W1a, b1a, W1b, b1b, W2a, b2a, W2b, b2b):
    src = edge_index[0]
    dst = edge_index[1]
    pad = E_PAD - src.shape[0]
    srcs = jnp.concatenate([src, jnp.zeros((pad,), jnp.int32)])
    dsts = jnp.concatenate([dst, jnp.full((pad,), N_NODES, jnp.int32)])
    srcs = srcs.reshape(NC * NS, C_GROUPS, GSZ, K_EDGE)
    dsts = dsts.reshape(NC * NS, C_GROUPS, GSZ, K_EDGE)
    srcs = jnp.pad(srcs, ((0, 0), (0, 1), (0, 0), (0, 0)))
    dsts = jnp.pad(dsts, ((0, 0), (0, 1), (0, 0), (0, 0)))
    srcs = srcs.reshape(NC * NS * NGRP_PAD, GSZ, K_EDGE)
    dsts = dsts.reshape(NC * NS * NGRP_PAD, GSZ, K_EDGE)

    x_pad = jnp.pad(x, ((0, N_PAD - N_NODES), (0, 0)))

    b1a_ = b1a.reshape(1, DIM)
    b1b_ = b1b.reshape(1, DIM)
    b2a_ = b2a.reshape(1, DIM)
    b2b_ = b2b.reshape(1, DIM)

    p1 = _sc_aggregate(x_pad, srcs, dsts)
    h = _tc_mlp(p1, x_pad, W1a, b1a_, W1b, b1b_, final_relu=True)
    p2 = _sc_aggregate(h, srcs, dsts)
    out = _tc_mlp(p2, h, W2a, b2a_, W2b, b2b_, final_relu=False)
    return out[:N_NODES]


# speedup vs baseline: 1.1253x; 1.1253x over previous
import functools

import jax
import jax.numpy as jnp
from jax import lax
from jax.experimental import pallas as pl
from jax.experimental.pallas import tpu as pltpu
from jax.experimental.pallas import tpu_sc as plsc

N_NODES = 10000
DIM = 128
N_PAD = 10240
K_EDGE = 64
C_CHUNKS = 160
NC, NS = 2, 16
E_PAD = NC * NS * C_CHUNKS * K_EDGE
ROWS_PER_TILE = N_PAD // NS
GSZ = 32
C_GROUPS = C_CHUNKS // GSZ
NBUF = 5


def _sc_aggregate_body(h_hbm, srcs_hbm, dsts_hbm, out_hbm,
                       sidx, didx, rows, acc, sem_g, sem_s):
    c = lax.axis_index("c")
    s = lax.axis_index("s")
    row0 = pl.multiple_of(s * ROWS_PER_TILE, ROWS_PER_TILE)
    wid = c * NS + s

    pltpu.sync_copy(h_hbm.at[pl.ds(row0, ROWS_PER_TILE)],
                    acc.at[pl.ds(row0, ROWS_PER_TILE)])
    plsc.subcore_barrier()

    def wait_g():
        pltpu.make_async_copy(h_hbm.at[sidx.at[0]], rows.at[0], sem_g).wait()

    def wait_s():
        pltpu.make_async_copy(rows.at[0], acc.at[pl.ds(0, K_EDGE)],
                              sem_s).wait()

    base = wid * C_CHUNKS

    def gather(j, b):
        pltpu.async_copy(h_hbm.at[sidx.at[j]], rows.at[b], sem_g)

    def scatter(j, b):
        pltpu.async_copy(rows.at[b], acc.at[didx.at[j]], sem_s, add=True)

    def supergroup(sg, _):
        g0 = pl.multiple_of(base + sg * GSZ, 8)
        pltpu.sync_copy(srcs_hbm.at[pl.ds(g0, GSZ)], sidx)
        pltpu.sync_copy(dsts_hbm.at[pl.ds(g0, GSZ)], didx)
        gather(0, 0)
        gather(1, 1)
        gather(2, 2)
        for j in range(GSZ):
            b = j % NBUF
            wait_g()
            scatter(j, b)
            if j + 3 < GSZ:
                if j >= 2:
                    wait_s()
                gather(j + 3, (j + 3) % NBUF)
        for _i in range(5):
            wait_s()
        return 0

    lax.fori_loop(0, C_GROUPS, supergroup, 0)
    plsc.subcore_barrier()

    pltpu.sync_copy(acc.at[pl.ds(row0, ROWS_PER_TILE)],
                    out_hbm.at[c, pl.ds(row0, ROWS_PER_TILE)])


_sc_aggregate = functools.partial(
    pl.kernel,
    out_type=jax.ShapeDtypeStruct((NC, N_PAD, DIM), jnp.float32),
    mesh=plsc.VectorSubcoreMesh(core_axis_name="c", subcore_axis_name="s"),
    scratch_types=[
        pltpu.VMEM((GSZ, K_EDGE), jnp.int32),
        pltpu.VMEM((GSZ, K_EDGE), jnp.int32),
        pltpu.VMEM((NBUF, K_EDGE, DIM), jnp.float32),
        pltpu.VMEM_SHARED((N_PAD, DIM), jnp.float32),
        pltpu.SemaphoreType.DMA,
        pltpu.SemaphoreType.DMA,
    ],
)(_sc_aggregate_body)


def _mlp_body(final_relu, p_ref, h_ref, wa_ref, ba_ref, wb_ref, bb_ref, o_ref):
    z = p_ref[0] + p_ref[1] - h_ref[...]
    y = jnp.maximum(
        jnp.dot(z, wa_ref[...], preferred_element_type=jnp.float32)
        + ba_ref[...], 0.0)
    y = jnp.dot(y, wb_ref[...], preferred_element_type=jnp.float32) + bb_ref[...]
    if final_relu:
        y = jnp.maximum(y, 0.0)
    o_ref[...] = y


def _tc_mlp(p, h, wa, ba, wb, bb, final_relu):
    rb = 1280
    grid = N_PAD // rb
    return pl.pallas_call(
        functools.partial(_mlp_body, final_relu),
        grid=(grid,),
        in_specs=[
            pl.BlockSpec((NC, rb, DIM), lambda i: (0, i, 0)),
            pl.BlockSpec((rb, DIM), lambda i: (i, 0)),
            pl.BlockSpec((DIM, DIM), lambda i: (0, 0)),
            pl.BlockSpec((1, DIM), lambda i: (0, 0)),
            pl.BlockSpec((DIM, DIM), lambda i: (0, 0)),
            pl.BlockSpec((1, DIM), lambda i: (0, 0)),
        ],
        out_specs=pl.BlockSpec((rb, DIM), lambda i: (i, 0)),
        out_shape=jax.ShapeDtypeStruct((N_PAD, DIM), jnp.float32),
    )(p, h, wa, ba, wb, bb)


def kernel(x, edge_index, W1a, b1a, W1b, b1b, W2a, b2a, W2b, b2b):
    src = edge_index[0]
    dst = edge_index[1]
    pad = E_PAD - src.shape[0]
    srcs = jnp.concatenate([src, jnp.zeros((pad,), jnp.int32)])
    dsts = jnp.concatenate([dst, jnp.full((pad,), N_NODES, jnp.int32)])
    srcs = srcs.reshape(NC * NS * C_CHUNKS, K_EDGE)
    dsts = dsts.reshape(NC * NS * C_CHUNKS, K_EDGE)

    x_pad = jnp.pad(x, ((0, N_PAD - N_NODES), (0, 0)))

    b1a_ = b1a.reshape(1, DIM)
    b1b_ = b1b.reshape(1, DIM)
    b2a_ = b2a.reshape(1, DIM)
    b2b_ = b2b.reshape(1, DIM)

    p1 = _sc_aggregate(x_pad, srcs, dsts)
    h = _tc_mlp(p1, x_pad, W1a, b1a_, W1b, b1b_, final_relu=True)
    p2 = _sc_aggregate(h, srcs, dsts)
    out = _tc_mlp(p2, h, W2a, b2a_, W2b, b2b_, final_relu=False)
    return out[:N_NODES]
